# baseline (speedup 1.0000x reference)
import functools
import jax
import jax.numpy as jnp
from jax import lax
from jax.experimental import pallas as pl
from jax.experimental.pallas import tpu as pltpu
from jax.experimental.pallas import tpu_sc as plsc

M, B, D, C = 8, 16384, 128, 10
CP = 16

NC, NS, L = 2, 16, 16
NW = NC * NS
S = B // NW
RPT = S * C // 128


def _tc_votes_body(x_ref, w_ref, b_ref, votes_ref, w4_s):
    bs = x_ref.shape[0]

    @pl.when(pl.program_id(0) == 0)
    def _pack():
        zpad = jnp.zeros((CP - C, D), jnp.float32)
        pieces = []
        for m in range(M):
            pieces.append(w_ref[m].T)
            pieces.append(zpad)
        w4_s[:] = jnp.concatenate(pieces, axis=0)

    xT = x_ref[:].T
    logitsT = jnp.dot(w4_s[:], xT, preferred_element_type=jnp.float32)
    logitsT = logitsT + b_ref[:]
    l3 = logitsT.reshape(M, CP, bs)[:, :C, :]
    mx = jnp.max(l3, axis=1, keepdims=True)
    iota = lax.broadcasted_iota(jnp.int32, (M, C, bs), 1)
    cand = jnp.where(l3 >= mx, iota, C)
    votes_ref[:] = jnp.min(cand, axis=1)


def _sc_hist_body(votes_hbm, out_hbm, votes_v, counts_v):
    wid = lax.axis_index("s") * NC + lax.axis_index("c")
    base = wid * S
    pltpu.sync_copy(votes_hbm.at[:, pl.ds(base, S)], votes_v)

    ones = jnp.ones((L,), jnp.float32)
    zerosf = jnp.zeros((L,), jnp.float32)
    lane = lax.iota(jnp.int32, L)

    def zero_row(r, carry):
        for j in range(128 // L):
            counts_v[r, pl.ds(j * L, L)] = zerosf
        return carry

    lax.fori_loop(0, RPT, zero_row, 0)

    def hist_body(i, carry):
        sbase = i * L
        flat = (lane + sbase) * C
        for m in range(M):
            v = votes_v[m, pl.ds(sbase, L)]
            f = flat + v
            plsc.addupdate_scatter(
                counts_v,
                [lax.shift_right_logical(f, 7),
                 lax.bitwise_and(f, jnp.full((L,), 127, jnp.int32))],
                ones)
        return carry

    lax.fori_loop(0, S // L, hist_body, 0)
    pltpu.sync_copy(counts_v, out_hbm.at[pl.ds(wid * RPT, RPT)])


def kernel(x, W, b):
    b4 = jnp.pad(b, ((0, 0), (0, CP - C))).reshape(M * CP, 1)
    bs = 4096
    votes = pl.pallas_call(
        _tc_votes_body,
        grid=(B // bs,),
        in_specs=[
            pl.BlockSpec((bs, D), lambda i: (i, 0)),
            pl.BlockSpec((M, D, C), lambda i: (0, 0, 0)),
            pl.BlockSpec((M * CP, 1), lambda i: (0, 0)),
        ],
        out_specs=pl.BlockSpec((M, bs), lambda i: (0, i)),
        out_shape=jax.ShapeDtypeStruct((M, B), jnp.int32),
        scratch_shapes=[pltpu.VMEM((M * CP, D), jnp.float32)],
    )(x, W, b4)

    mesh = plsc.VectorSubcoreMesh(core_axis_name="c", subcore_axis_name="s")
    sc_hist = functools.partial(
        pl.kernel,
        mesh=mesh,
        compiler_params=pltpu.CompilerParams(
            needs_layout_passes=False, skip_device_barrier=True),
        out_type=jax.ShapeDtypeStruct((B * C // 128, 128), jnp.float32),
        scratch_types=[
            pltpu.VMEM((M, S), jnp.int32),
            pltpu.VMEM((RPT, 128), jnp.float32),
        ],
    )(_sc_hist_body)
    counts2d = sc_hist(votes)
    return counts2d.reshape(B, C)


# speedup vs baseline: 1.1825x; 1.1825x over previous
import functools
import jax
import jax.numpy as jnp
from jax import lax
from jax.experimental import pallas as pl
from jax.experimental.pallas import tpu as pltpu
from jax.experimental.pallas import tpu_sc as plsc

M, B, D, C = 8, 16384, 128, 10
CP = 16

NC, NS, L = 2, 16, 16
NW = NC * NS
S = B // NW
RPT = S * C // 128


def _tc_votes_body(x_ref, w_ref, b_ref, votes_ref, w4_s):
    bs = x_ref.shape[0]

    @pl.when(pl.program_id(0) == 0)
    def _pack():
        zpad = jnp.zeros((CP - C, D), jnp.float32)
        pieces = []
        for m in range(M):
            pieces.append(w_ref[m].T)
            pieces.append(zpad)
        w4_s[:] = jnp.concatenate(pieces, axis=0)

    xT = x_ref[:].T
    logitsT = jnp.dot(w4_s[:], xT, preferred_element_type=jnp.float32)
    l3 = logitsT.reshape(M, CP, bs)[:, :C, :]
    l3 = l3 + b_ref[:][:, :, None]
    mx = jnp.max(l3, axis=1, keepdims=True)
    iota = lax.broadcasted_iota(jnp.int32, (M, C, bs), 1)
    cand = jnp.where(l3 >= mx, iota, C)
    votes_ref[:] = jnp.min(cand, axis=1)


def _sc_hist_body(votes_hbm, out_hbm, votes_v, counts_v):
    wid = lax.axis_index("s") * NC + lax.axis_index("c")
    base = wid * S
    pltpu.sync_copy(votes_hbm.at[:, pl.ds(base, S)], votes_v)

    ones = jnp.ones((L,), jnp.float32)
    zerosf = jnp.zeros((L,), jnp.float32)
    lane = lax.iota(jnp.int32, L)

    def hist_body(i, carry):
        sbase = i * L
        samp = lane + sbase

        def zero_c(c, carry2):
            plsc.store_scatter(counts_v, [samp, jnp.full((L,), 1, jnp.int32) * c],
                               zerosf)
            return carry2

        lax.fori_loop(0, C, zero_c, 0)

        def add_m(m, carry2):
            v = votes_v[m, pl.ds(sbase, L)]
            plsc.addupdate_scatter(counts_v, [samp, v], ones)
            return carry2

        lax.fori_loop(0, M, add_m, 0)
        return carry

    lax.fori_loop(0, S // L, hist_body, 0)
    pltpu.sync_copy(counts_v, out_hbm.at[pl.ds(base, S)])


def kernel(x, W, b):
    bs = 4096
    votes = pl.pallas_call(
        _tc_votes_body,
        grid=(B // bs,),
        in_specs=[
            pl.BlockSpec((bs, D), lambda i: (i, 0)),
            pl.BlockSpec((M, D, C), lambda i: (0, 0, 0)),
            pl.BlockSpec((M, C), lambda i: (0, 0)),
        ],
        out_specs=pl.BlockSpec((M, bs), lambda i: (0, i)),
        out_shape=jax.ShapeDtypeStruct((M, B), jnp.int32),
        scratch_shapes=[pltpu.VMEM((M * CP, D), jnp.float32)],
    )(x, W, b)

    mesh = plsc.VectorSubcoreMesh(core_axis_name="c", subcore_axis_name="s")
    sc_hist = functools.partial(
        pl.kernel,
        mesh=mesh,
        compiler_params=pltpu.CompilerParams(
            needs_layout_passes=False, skip_device_barrier=True),
        out_type=jax.ShapeDtypeStruct((B, C), jnp.float32),
        scratch_types=[
            pltpu.VMEM((M, S), jnp.int32),
            pltpu.VMEM((S, C), jnp.float32),
        ],
    )(_sc_hist_body)
    return sc_hist(votes)


# speedup vs baseline: 1.1890x; 1.0054x over previous
import functools
import jax
import jax.numpy as jnp
from jax import lax
from jax.experimental import pallas as pl
from jax.experimental.pallas import tpu as pltpu
from jax.experimental.pallas import tpu_sc as plsc

M, B, D, C = 8, 16384, 128, 10
CP = 16

NC, NS, L = 2, 16, 16
NW = NC * NS
S = B // NW
RPT = S * C // 128


def _tc_votes_body(x_ref, w_ref, b_ref, votes_ref, w4_s):
    bs = x_ref.shape[0]

    @pl.when(pl.program_id(0) == 0)
    def _pack():
        zpad = jnp.zeros((CP - C, D), jnp.float32)
        pieces = []
        for m in range(M):
            pieces.append(w_ref[m].T)
            pieces.append(zpad)
        w4_s[:] = jnp.concatenate(pieces, axis=0)

    xT = x_ref[:].T
    logitsT = jnp.dot(w4_s[:], xT, preferred_element_type=jnp.float32)
    l3 = logitsT.reshape(M, CP, bs)[:, :C, :]
    l3 = l3 + b_ref[:][:, :, None]
    mx = jnp.max(l3, axis=1, keepdims=True)
    iota = lax.broadcasted_iota(jnp.int32, (M, C, bs), 1)
    cand = jnp.where(l3 >= mx, iota, C)
    votes_ref[:] = jnp.min(cand, axis=1)


def _sc_hist_body(votes_hbm, out_hbm, votes_v, counts_v):
    wid = lax.axis_index("s") * NC + lax.axis_index("c")
    base = wid * S
    pltpu.sync_copy(votes_hbm.at[:, pl.ds(base, S)], votes_v)

    ones = jnp.ones((L,), jnp.float32)
    zerosf = jnp.zeros((L,), jnp.float32)
    lane = lax.iota(jnp.int32, L)

    def hist_body(i, carry):
        sbase = i * L
        samp = lane + sbase
        for c in range(C):
            plsc.store_scatter(counts_v, [samp, jnp.full((L,), c, jnp.int32)],
                               zerosf)
        for m in range(M):
            v = votes_v[m, pl.ds(sbase, L)]
            plsc.addupdate_scatter(counts_v, [samp, v], ones)
        return carry

    lax.fori_loop(0, S // L, hist_body, 0)
    pltpu.sync_copy(counts_v, out_hbm.at[pl.ds(base, S)])


def kernel(x, W, b):
    bs = 4096
    votes = pl.pallas_call(
        _tc_votes_body,
        grid=(B // bs,),
        in_specs=[
            pl.BlockSpec((bs, D), lambda i: (i, 0)),
            pl.BlockSpec((M, D, C), lambda i: (0, 0, 0)),
            pl.BlockSpec((M, C), lambda i: (0, 0)),
        ],
        out_specs=pl.BlockSpec((M, bs), lambda i: (0, i)),
        out_shape=jax.ShapeDtypeStruct((M, B), jnp.int32),
        scratch_shapes=[pltpu.VMEM((M * CP, D), jnp.float32)],
    )(x, W, b)

    mesh = plsc.VectorSubcoreMesh(core_axis_name="c", subcore_axis_name="s")
    sc_hist = functools.partial(
        pl.kernel,
        mesh=mesh,
        compiler_params=pltpu.CompilerParams(
            needs_layout_passes=False, skip_device_barrier=True),
        out_type=jax.ShapeDtypeStruct((B, C), jnp.float32),
        scratch_types=[
            pltpu.VMEM((M, S), jnp.int32),
            pltpu.VMEM((S, C), jnp.float32),
        ],
    )(_sc_hist_body)
    return sc_hist(votes)
